# baseline (speedup 1.0000x reference)
import functools

import jax
import jax.numpy as jnp
from jax.experimental import pallas as pl
from jax.experimental.pallas import tpu as pltpu

_B_BLK = 8


def _conv_stack_kernel(x_ref, s1_ref, b1_ref, s2_ref, b2_ref, s3_ref, b3_ref,
                       o_ref):
    B = x_ref.shape[0]
    x = x_ref[...]

    xr = x.reshape(B, 4, 21, 4, 84)
    pieces = []
    for i in range(8):
        u, ph = i // 4, i % 4
        for c in range(4):
            s = xr[:, c, u:u + 20, ph, :]
            pieces.append(jnp.pad(s, ((0, 0), (0, 0), (0, 44))))
    p = jnp.concatenate(pieces, axis=-1)
    a = jnp.dot(p.reshape(B * 20, 4096), s1_ref[...],
                preferred_element_type=jnp.float32)
    a = jnp.maximum(a + b1_ref[...], 0.0).reshape(B, 20, 640)

    ar = a.reshape(B, 10, 2, 640)
    pieces = [ar[:, u:u + 9, ph, :]
              for u in range(2) for ph in range(2)]
    p = jnp.concatenate(pieces, axis=-1)
    a = jnp.dot(p.reshape(B * 9, 2560), s2_ref[...],
                preferred_element_type=jnp.float32)
    a = jnp.maximum(a + b2_ref[...], 0.0).reshape(B, 9, 576)

    pieces = [jnp.pad(a[:, i:i + 7, :], ((0, 0), (0, 0), (0, 64)))
              for i in range(3)]
    p = jnp.concatenate(pieces, axis=-1)
    a = jnp.dot(p.reshape(B * 7, 1920), s3_ref[...],
                preferred_element_type=jnp.float32)
    a = jnp.maximum(a + b3_ref[...], 0.0)
    o_ref[...] = a.reshape(B, 7, 448)


def _fc_head_kernel(x_ref, w1_ref, b1_ref, w2_ref, b2_ref, o_ref):
    h = jnp.dot(x_ref[...], w1_ref[...], preferred_element_type=jnp.float32)
    h = jnp.maximum(h + b1_ref[...], 0.0)
    o = jnp.dot(h, w2_ref[...], preferred_element_type=jnp.float32)
    o_ref[...] = o + b2_ref[...]


def _window_mask(n_in, n_out, k, stride):
    a = jnp.arange(n_in)[:, None, None]
    t = jnp.arange(n_out)[None, :, None]
    j = jnp.arange(k)[None, None, :]
    return (a == stride * t + j).astype(jnp.float32)


def kernel(c1_w, c1_b, c2_w, c2_b, c3_w, c3_b, fc1_w, fc1_b, fc2_w, fc2_b,
           x_nchw):
    N = x_nchw.shape[0]
    x = x_nchw.astype(jnp.float32)

    w1r = c1_w.reshape(8, 8, 4, 32)
    s1 = jnp.einsum("wtj,ijco->icwto", _window_mask(128, 20, 8, 4), w1r)
    s1 = s1.reshape(4096, 640)
    b1t = jnp.tile(c1_b, (1, 20))

    w2r = c2_w.reshape(4, 4, 32, 64)
    s2 = jnp.einsum("atj,ijco->iacto", _window_mask(20, 9, 4, 2), w2r)
    s2 = s2.reshape(2560, 576)
    b2t = jnp.tile(c2_b, (1, 9))

    w3r = c3_w.reshape(3, 3, 64, 64)
    s3 = jnp.einsum("atj,ijco->iacto", _window_mask(10, 7, 3, 1), w3r)
    s3 = s3.reshape(1920, 448)
    b3t = jnp.tile(c3_b, (1, 7))

    feat = pl.pallas_call(
        _conv_stack_kernel,
        out_shape=jax.ShapeDtypeStruct((N, 7, 448), jnp.float32),
        grid=(N // _B_BLK,),
        in_specs=[
            pl.BlockSpec((_B_BLK, 4, 84, 84), lambda i: (i, 0, 0, 0)),
            pl.BlockSpec((4096, 640), lambda i: (0, 0)),
            pl.BlockSpec((1, 640), lambda i: (0, 0)),
            pl.BlockSpec((2560, 576), lambda i: (0, 0)),
            pl.BlockSpec((1, 576), lambda i: (0, 0)),
            pl.BlockSpec((1920, 448), lambda i: (0, 0)),
            pl.BlockSpec((1, 448), lambda i: (0, 0)),
        ],
        out_specs=pl.BlockSpec((_B_BLK, 7, 448), lambda i: (i, 0, 0)),
        compiler_params=pltpu.CompilerParams(
            dimension_semantics=("parallel",)),
    )(x, s1, b1t, s2, b2t, s3, b3t)

    feat = feat.reshape(N, 49 * 64)

    tm = 128
    return pl.pallas_call(
        _fc_head_kernel,
        out_shape=jax.ShapeDtypeStruct((N, 18), jnp.float32),
        grid=(N // tm,),
        in_specs=[
            pl.BlockSpec((tm, 3136), lambda i: (i, 0)),
            pl.BlockSpec((3136, 512), lambda i: (0, 0)),
            pl.BlockSpec((1, 512), lambda i: (0, 0)),
            pl.BlockSpec((512, 18), lambda i: (0, 0)),
            pl.BlockSpec((1, 18), lambda i: (0, 0)),
        ],
        out_specs=pl.BlockSpec((tm, 18), lambda i: (i, 0)),
        compiler_params=pltpu.CompilerParams(
            dimension_semantics=("parallel",)),
    )(feat, fc1_w, fc1_b, fc2_w, fc2_b)


# speedup vs baseline: 1.0529x; 1.0529x over previous
import functools

import jax
import jax.numpy as jnp
from jax.experimental import pallas as pl
from jax.experimental.pallas import tpu as pltpu

_B_BLK = 16


def _conv_stack_kernel(x_ref, s1_ref, b1_ref, s2_ref, b2_ref, s3_ref, b3_ref,
                       o_ref):
    B = x_ref.shape[0]
    x = x_ref[...]

    xr = x.reshape(B, 4, 21, 4, 84)
    pieces = []
    for i in range(8):
        u, ph = i // 4, i % 4
        for c in range(4):
            s = xr[:, c, u:u + 20, ph, :]
            pieces.append(jnp.pad(s, ((0, 0), (0, 0), (0, 44))))
    p = jnp.concatenate(pieces, axis=-1)
    a = jnp.dot(p.reshape(B * 20, 4096), s1_ref[...],
                preferred_element_type=jnp.float32)
    a = jnp.maximum(a + b1_ref[...], 0.0).reshape(B, 20, 640)

    ar = a.reshape(B, 10, 2, 640)
    pieces = [ar[:, u:u + 9, ph, :]
              for u in range(2) for ph in range(2)]
    p = jnp.concatenate(pieces, axis=-1)
    a = jnp.dot(p.reshape(B * 9, 2560), s2_ref[...],
                preferred_element_type=jnp.float32)
    a = jnp.maximum(a + b2_ref[...], 0.0).reshape(B, 9, 576)

    pieces = [jnp.pad(a[:, i:i + 7, :], ((0, 0), (0, 0), (0, 64)))
              for i in range(3)]
    p = jnp.concatenate(pieces, axis=-1)
    a = jnp.dot(p.reshape(B * 7, 1920), s3_ref[...],
                preferred_element_type=jnp.float32)
    a = jnp.maximum(a + b3_ref[...], 0.0)
    o_ref[...] = a.reshape(B, 7, 448)


def _fc_head_kernel(x_ref, w1_ref, b1_ref, w2_ref, b2_ref, o_ref):
    h = jnp.dot(x_ref[...], w1_ref[...], preferred_element_type=jnp.float32)
    h = jnp.maximum(h + b1_ref[...], 0.0)
    o = jnp.dot(h, w2_ref[...], preferred_element_type=jnp.float32)
    o_ref[...] = o + b2_ref[...]


def _sel_weight(w_ijco, n_in, n_out, stride, channel_minor=True):
    kh, k, C, O = w_ijco.shape
    a = jnp.arange(n_in)[:, None, None]
    t = jnp.arange(n_out)[None, :, None]
    j = jnp.arange(k)[None, None, :]
    m = (a == stride * t + j).astype(jnp.float32)
    w_icjo = jnp.transpose(w_ijco, (0, 2, 1, 3))
    if channel_minor:
        prod = (m[None, :, None, :, :, None]
                * w_icjo[:, None, :, None, :, :])
    else:
        prod = (m[None, None, :, :, :, None]
                * w_icjo[:, :, None, None, :, :])
    s = prod.sum(axis=4)
    return s.reshape(kh * C * n_in, n_out * O)


def kernel(c1_w, c1_b, c2_w, c2_b, c3_w, c3_b, fc1_w, fc1_b, fc2_w, fc2_b,
           x_nchw):
    N = x_nchw.shape[0]
    x = x_nchw.astype(jnp.float32)

    s1 = _sel_weight(c1_w.reshape(8, 8, 4, 32), 128, 20, 4,
                     channel_minor=False)
    b1t = jnp.tile(c1_b, (1, 20))
    s2 = _sel_weight(c2_w.reshape(4, 4, 32, 64), 20, 9, 2)
    b2t = jnp.tile(c2_b, (1, 9))
    s3 = _sel_weight(c3_w.reshape(3, 3, 64, 64), 10, 7, 1)
    b3t = jnp.tile(c3_b, (1, 7))

    feat = pl.pallas_call(
        _conv_stack_kernel,
        out_shape=jax.ShapeDtypeStruct((N, 7, 448), jnp.float32),
        grid=(N // _B_BLK,),
        in_specs=[
            pl.BlockSpec((_B_BLK, 4, 84, 84), lambda i: (i, 0, 0, 0)),
            pl.BlockSpec((4096, 640), lambda i: (0, 0)),
            pl.BlockSpec((1, 640), lambda i: (0, 0)),
            pl.BlockSpec((2560, 576), lambda i: (0, 0)),
            pl.BlockSpec((1, 576), lambda i: (0, 0)),
            pl.BlockSpec((1920, 448), lambda i: (0, 0)),
            pl.BlockSpec((1, 448), lambda i: (0, 0)),
        ],
        out_specs=pl.BlockSpec((_B_BLK, 7, 448), lambda i: (i, 0, 0)),
        compiler_params=pltpu.CompilerParams(
            dimension_semantics=("parallel",)),
    )(x, s1, b1t, s2, b2t, s3, b3t)

    feat = feat.reshape(N, 49 * 64)

    tm = 128
    return pl.pallas_call(
        _fc_head_kernel,
        out_shape=jax.ShapeDtypeStruct((N, 18), jnp.float32),
        grid=(N // tm,),
        in_specs=[
            pl.BlockSpec((tm, 3136), lambda i: (i, 0)),
            pl.BlockSpec((3136, 512), lambda i: (0, 0)),
            pl.BlockSpec((1, 512), lambda i: (0, 0)),
            pl.BlockSpec((512, 18), lambda i: (0, 0)),
            pl.BlockSpec((1, 18), lambda i: (0, 0)),
        ],
        out_specs=pl.BlockSpec((tm, 18), lambda i: (i, 0)),
        compiler_params=pltpu.CompilerParams(
            dimension_semantics=("parallel",)),
    )(feat, fc1_w, fc1_b, fc2_w, fc2_b)


# speedup vs baseline: 1.0691x; 1.0154x over previous
import functools

import jax
import jax.numpy as jnp
from jax.experimental import pallas as pl
from jax.experimental.pallas import tpu as pltpu

_B_BLK = 16


def _conv_stack_kernel(x_ref, s1_ref, b1_ref, s2_ref, b2_ref, s3_ref, b3_ref,
                       o_ref):
    B = x_ref.shape[0]
    x = x_ref[...].astype(jnp.bfloat16)

    xr = x.reshape(B, 4, 21, 4, 84)
    pieces = []
    for i in range(8):
        u, ph = i // 4, i % 4
        for c in range(4):
            s = xr[:, c, u:u + 20, ph, :]
            pieces.append(jnp.pad(s, ((0, 0), (0, 0), (0, 44))))
    p = jnp.concatenate(pieces, axis=-1)
    a = jnp.dot(p.reshape(B * 20, 4096), s1_ref[...],
                preferred_element_type=jnp.float32)
    a = jnp.maximum(a + b1_ref[...], 0.0)
    a = a.astype(jnp.bfloat16).reshape(B, 20, 640)

    ar = a.reshape(B, 10, 2, 640)
    pieces = [ar[:, u:u + 9, ph, :]
              for u in range(2) for ph in range(2)]
    p = jnp.concatenate(pieces, axis=-1)
    a = jnp.dot(p.reshape(B * 9, 2560), s2_ref[...],
                preferred_element_type=jnp.float32)
    a = jnp.maximum(a + b2_ref[...], 0.0)
    a = a.astype(jnp.bfloat16).reshape(B, 9, 576)

    pieces = [jnp.pad(a[:, i:i + 7, :], ((0, 0), (0, 0), (0, 64)))
              for i in range(3)]
    p = jnp.concatenate(pieces, axis=-1)
    a = jnp.dot(p.reshape(B * 7, 1920), s3_ref[...],
                preferred_element_type=jnp.float32)
    a = jnp.maximum(a + b3_ref[...], 0.0)
    o_ref[...] = a.reshape(B, 7, 448).astype(o_ref.dtype)


def _fc_head_kernel(x_ref, w1_ref, b1_ref, w2_ref, b2_ref, o_ref):
    h = jnp.dot(x_ref[...], w1_ref[...], preferred_element_type=jnp.float32)
    h = jnp.maximum(h + b1_ref[...], 0.0).astype(jnp.bfloat16)
    o = jnp.dot(h, w2_ref[...], preferred_element_type=jnp.float32)
    o_ref[...] = o + b2_ref[...]


def _sel_weight(w_ijco, n_in, n_out, stride, channel_minor=True):
    kh, k, C, O = w_ijco.shape
    a = jnp.arange(n_in)[:, None, None]
    t = jnp.arange(n_out)[None, :, None]
    j = jnp.arange(k)[None, None, :]
    m = (a == stride * t + j).astype(jnp.float32)
    w_icjo = jnp.transpose(w_ijco, (0, 2, 1, 3))
    if channel_minor:
        prod = (m[None, :, None, :, :, None]
                * w_icjo[:, None, :, None, :, :])
    else:
        prod = (m[None, None, :, :, :, None]
                * w_icjo[:, :, None, None, :, :])
    s = prod.sum(axis=4)
    return s.reshape(kh * C * n_in, n_out * O).astype(jnp.bfloat16)


def kernel(c1_w, c1_b, c2_w, c2_b, c3_w, c3_b, fc1_w, fc1_b, fc2_w, fc2_b,
           x_nchw):
    N = x_nchw.shape[0]
    x = x_nchw.astype(jnp.float32)

    s1 = _sel_weight(c1_w.reshape(8, 8, 4, 32), 128, 20, 4,
                     channel_minor=False)
    b1t = jnp.tile(c1_b, (1, 20))
    s2 = _sel_weight(c2_w.reshape(4, 4, 32, 64), 20, 9, 2)
    b2t = jnp.tile(c2_b, (1, 9))
    s3 = _sel_weight(c3_w.reshape(3, 3, 64, 64), 10, 7, 1)
    b3t = jnp.tile(c3_b, (1, 7))

    feat = pl.pallas_call(
        _conv_stack_kernel,
        out_shape=jax.ShapeDtypeStruct((N, 7, 448), jnp.bfloat16),
        grid=(N // _B_BLK,),
        in_specs=[
            pl.BlockSpec((_B_BLK, 4, 84, 84), lambda i: (i, 0, 0, 0)),
            pl.BlockSpec((4096, 640), lambda i: (0, 0)),
            pl.BlockSpec((1, 640), lambda i: (0, 0)),
            pl.BlockSpec((2560, 576), lambda i: (0, 0)),
            pl.BlockSpec((1, 576), lambda i: (0, 0)),
            pl.BlockSpec((1920, 448), lambda i: (0, 0)),
            pl.BlockSpec((1, 448), lambda i: (0, 0)),
        ],
        out_specs=pl.BlockSpec((_B_BLK, 7, 448), lambda i: (i, 0, 0)),
        compiler_params=pltpu.CompilerParams(
            dimension_semantics=("arbitrary",)),
    )(x, s1, b1t, s2, b2t, s3, b3t)

    feat = feat.reshape(N, 49 * 64)

    tm = 128
    return pl.pallas_call(
        _fc_head_kernel,
        out_shape=jax.ShapeDtypeStruct((N, 18), jnp.float32),
        grid=(N // tm,),
        in_specs=[
            pl.BlockSpec((tm, 3136), lambda i: (i, 0)),
            pl.BlockSpec((3136, 512), lambda i: (0, 0)),
            pl.BlockSpec((1, 512), lambda i: (0, 0)),
            pl.BlockSpec((512, 18), lambda i: (0, 0)),
            pl.BlockSpec((1, 18), lambda i: (0, 0)),
        ],
        out_specs=pl.BlockSpec((tm, 18), lambda i: (i, 0)),
        compiler_params=pltpu.CompilerParams(
            dimension_semantics=("arbitrary",)),
    )(feat, fc1_w.astype(jnp.bfloat16), fc1_b,
      fc2_w.astype(jnp.bfloat16), fc2_b)


# speedup vs baseline: 1.7338x; 1.6217x over previous
import functools

import jax
import jax.numpy as jnp
from jax.experimental import pallas as pl
from jax.experimental.pallas import tpu as pltpu

_B_LANES = 128


def _net_kernel(x_ref, w1_ref, b1_ref, w2_ref, b2_ref, w3_ref, b3_ref,
                f1_ref, fb1_ref, f2_ref, fb2_ref, o_ref):
    xb = x_ref[...].astype(jnp.bfloat16)

    rows = []
    for oh in range(20):
        slab = xb[4 * oh:4 * oh + 8].reshape(8 * 84 * 4, _B_LANES)
        acc = jnp.dot(w1_ref[...], slab, preferred_element_type=jnp.float32)
        rows.append(jnp.maximum(acc + b1_ref[...], 0.0).astype(jnp.bfloat16))
    a = jnp.stack(rows)

    rows = []
    for oh in range(9):
        slab = a[2 * oh:2 * oh + 4].reshape(4 * 640, _B_LANES)
        acc = jnp.dot(w2_ref[...], slab, preferred_element_type=jnp.float32)
        rows.append(jnp.maximum(acc + b2_ref[...], 0.0).astype(jnp.bfloat16))
    a = jnp.stack(rows)

    rows = []
    for oh in range(7):
        slab = a[oh:oh + 3].reshape(3 * 576, _B_LANES)
        acc = jnp.dot(w3_ref[...], slab, preferred_element_type=jnp.float32)
        rows.append(jnp.maximum(acc + b3_ref[...], 0.0).astype(jnp.bfloat16))
    feat = jnp.stack(rows).reshape(7 * 448, _B_LANES)

    h = jax.lax.dot_general(f1_ref[...], feat, (((0,), (0,)), ((), ())),
                            preferred_element_type=jnp.float32)
    h = jnp.maximum(h + fb1_ref[...], 0.0).astype(jnp.bfloat16)
    o = jax.lax.dot_general(f2_ref[...], h, (((0,), (0,)), ((), ())),
                            preferred_element_type=jnp.float32)
    o_ref[...] = o + fb2_ref[...]


def _sel_weight_lhs(w_ijco, n_in, n_out, stride):
    kh, k, C, O = w_ijco.shape
    a = jnp.arange(n_in)[:, None, None]
    t = jnp.arange(n_out)[None, :, None]
    j = jnp.arange(k)[None, None, :]
    m = (a == stride * t + j).astype(jnp.float32)
    m_b = jnp.transpose(m, (1, 0, 2))[:, None, None, :, None, :]
    w_b = jnp.transpose(w_ijco, (3, 0, 2, 1))[None, :, :, None, :, :]
    s = (m_b * w_b).sum(axis=-1)
    return s.reshape(n_out * O, kh * n_in * C).astype(jnp.bfloat16)


def kernel(c1_w, c1_b, c2_w, c2_b, c3_w, c3_b, fc1_w, fc1_b, fc2_w, fc2_b,
           x_nchw):
    N = x_nchw.shape[0]
    xt = jnp.transpose(x_nchw.astype(jnp.float32), (2, 3, 1, 0))

    w1 = _sel_weight_lhs(c1_w.reshape(8, 8, 4, 32), 84, 20, 4)
    w2 = _sel_weight_lhs(c2_w.reshape(4, 4, 32, 64), 20, 9, 2)
    w3 = _sel_weight_lhs(c3_w.reshape(3, 3, 64, 64), 9, 7, 1)
    b1 = jnp.tile(c1_b.reshape(-1), 20).reshape(640, 1)
    b2 = jnp.tile(c2_b.reshape(-1), 9).reshape(576, 1)
    b3 = jnp.tile(c3_b.reshape(-1), 7).reshape(448, 1)
    fb1 = fc1_b.reshape(512, 1)
    fb2 = fc2_b.reshape(18, 1)

    out = pl.pallas_call(
        _net_kernel,
        out_shape=jax.ShapeDtypeStruct((18, N), jnp.float32),
        grid=(N // _B_LANES,),
        in_specs=[
            pl.BlockSpec((84, 84, 4, _B_LANES), lambda g: (0, 0, 0, g)),
            pl.BlockSpec((640, 2688), lambda g: (0, 0)),
            pl.BlockSpec((640, 1), lambda g: (0, 0)),
            pl.BlockSpec((576, 2560), lambda g: (0, 0)),
            pl.BlockSpec((576, 1), lambda g: (0, 0)),
            pl.BlockSpec((448, 1728), lambda g: (0, 0)),
            pl.BlockSpec((448, 1), lambda g: (0, 0)),
            pl.BlockSpec((3136, 512), lambda g: (0, 0)),
            pl.BlockSpec((512, 1), lambda g: (0, 0)),
            pl.BlockSpec((512, 18), lambda g: (0, 0)),
            pl.BlockSpec((18, 1), lambda g: (0, 0)),
        ],
        out_specs=pl.BlockSpec((18, _B_LANES), lambda g: (0, g)),
        compiler_params=pltpu.CompilerParams(
            dimension_semantics=("arbitrary",)),
    )(xt, w1, b1, w2, b2, w3, b3,
      fc1_w.astype(jnp.bfloat16), fb1, fc2_w.astype(jnp.bfloat16), fb2)

    return out.T


# speedup vs baseline: 2.9203x; 1.6843x over previous
import functools

import jax
import jax.numpy as jnp
from jax.experimental import pallas as pl
from jax.experimental.pallas import tpu as pltpu

_B_LANES = 128


def _net_kernel(x_ref, w1_ref, b1_ref, w2_ref, b2_ref, w3_ref, b3_ref,
                f1_ref, fb1_ref, f2_ref, fb2_ref, o_ref):
    xb = x_ref[...].astype(jnp.bfloat16)

    rows = []
    for oh in range(20):
        slab = xb[4 * oh:4 * oh + 8].reshape(8 * 84 * 4, _B_LANES)
        acc = jnp.dot(w1_ref[...], slab, preferred_element_type=jnp.float32)
        rows.append(jnp.maximum(acc + b1_ref[...], 0.0).astype(jnp.bfloat16))
    a = jnp.stack(rows)

    rows = []
    for oh in range(9):
        slab = a[2 * oh:2 * oh + 4].reshape(4 * 640, _B_LANES)
        acc = jnp.dot(w2_ref[...], slab, preferred_element_type=jnp.float32)
        rows.append(jnp.maximum(acc + b2_ref[...], 0.0).astype(jnp.bfloat16))
    a = jnp.stack(rows)

    rows = []
    for oh in range(7):
        slab = a[oh:oh + 3].reshape(3 * 576, _B_LANES)
        acc = jnp.dot(w3_ref[...], slab, preferred_element_type=jnp.float32)
        rows.append(jnp.maximum(acc + b3_ref[...], 0.0).astype(jnp.bfloat16))
    feat = jnp.stack(rows).reshape(7 * 448, _B_LANES)

    h = jax.lax.dot_general(f1_ref[...], feat, (((0,), (0,)), ((), ())),
                            preferred_element_type=jnp.float32)
    h = jnp.maximum(h + fb1_ref[...], 0.0).astype(jnp.bfloat16)
    o = jax.lax.dot_general(f2_ref[...], h, (((0,), (0,)), ((), ())),
                            preferred_element_type=jnp.float32)
    o_ref[...] = o + fb2_ref[...]


def _sel_weight_lhs(w_ijco, n_in, n_out, stride):
    kh, k, C, O = w_ijco.shape
    base = jnp.transpose(w_ijco, (3, 0, 1, 2))
    base = jnp.pad(base, ((0, 0), (0, 0), (0, n_in - k), (0, 0)))
    base = base.reshape(O, kh * n_in * C).astype(jnp.bfloat16)
    rows = jnp.stack([jnp.roll(base, stride * C * t, axis=1)
                      for t in range(n_out)])
    return rows.reshape(n_out * O, kh * n_in * C)


def kernel(c1_w, c1_b, c2_w, c2_b, c3_w, c3_b, fc1_w, fc1_b, fc2_w, fc2_b,
           x_nchw):
    N = x_nchw.shape[0]
    xt = jnp.transpose(x_nchw.astype(jnp.float32), (2, 3, 1, 0))

    w1 = _sel_weight_lhs(c1_w.reshape(8, 8, 4, 32), 84, 20, 4)
    w2 = _sel_weight_lhs(c2_w.reshape(4, 4, 32, 64), 20, 9, 2)
    w3 = _sel_weight_lhs(c3_w.reshape(3, 3, 64, 64), 9, 7, 1)
    b1 = jnp.tile(c1_b.reshape(-1), 20).reshape(640, 1)
    b2 = jnp.tile(c2_b.reshape(-1), 9).reshape(576, 1)
    b3 = jnp.tile(c3_b.reshape(-1), 7).reshape(448, 1)
    fb1 = fc1_b.reshape(512, 1)
    fb2 = fc2_b.reshape(18, 1)

    out = pl.pallas_call(
        _net_kernel,
        out_shape=jax.ShapeDtypeStruct((18, N), jnp.float32),
        grid=(N // _B_LANES,),
        in_specs=[
            pl.BlockSpec((84, 84, 4, _B_LANES), lambda g: (0, 0, 0, g)),
            pl.BlockSpec((640, 2688), lambda g: (0, 0)),
            pl.BlockSpec((640, 1), lambda g: (0, 0)),
            pl.BlockSpec((576, 2560), lambda g: (0, 0)),
            pl.BlockSpec((576, 1), lambda g: (0, 0)),
            pl.BlockSpec((448, 1728), lambda g: (0, 0)),
            pl.BlockSpec((448, 1), lambda g: (0, 0)),
            pl.BlockSpec((3136, 512), lambda g: (0, 0)),
            pl.BlockSpec((512, 1), lambda g: (0, 0)),
            pl.BlockSpec((512, 18), lambda g: (0, 0)),
            pl.BlockSpec((18, 1), lambda g: (0, 0)),
        ],
        out_specs=pl.BlockSpec((18, _B_LANES), lambda g: (0, g)),
        compiler_params=pltpu.CompilerParams(
            dimension_semantics=("arbitrary",)),
    )(xt, w1, b1, w2, b2, w3, b3,
      fc1_w.astype(jnp.bfloat16), fb1, fc2_w.astype(jnp.bfloat16), fb2)

    return out.T
